# baseline (speedup 1.0000x reference)
import functools

import jax
import jax.numpy as jnp
from jax import lax
from jax.experimental import pallas as pl
from jax.experimental.pallas import tpu as pltpu
from jax.experimental.pallas import tpu_sc as plsc

_NC = 2
_NS = 16
_NW = _NC * _NS
_QW = 128
_UNROLL = 4


def _splat(v, q):
    idx = jnp.full((16,), q, jnp.int32)
    return v.at[idx].get(mode="promise_in_bounds")


def _nn_body(B, NT, qx_h, qy_h, qz_h, tx_h, ty_h, tz_h, out_h,
             qx_v, qy_v, qz_v, tx_v, ty_v, tz_v, res_v):
    wid = lax.axis_index("s") * _NC + lax.axis_index("c")
    iota = lax.iota(jnp.int32, 16)
    inf16 = jnp.full((16,), jnp.inf, jnp.float32)
    nchunk = NT // (16 * _UNROLL)

    for b in range(B):
        tbase = b * NT
        pltpu.sync_copy(tx_h.at[pl.ds(tbase, NT)], tx_v)
        pltpu.sync_copy(ty_h.at[pl.ds(tbase, NT)], ty_v)
        pltpu.sync_copy(tz_h.at[pl.ds(tbase, NT)], tz_v)
        qbase = b * (_QW * _NW) + wid * _QW
        pltpu.sync_copy(qx_h.at[pl.ds(qbase, _QW)], qx_v)
        pltpu.sync_copy(qy_h.at[pl.ds(qbase, _QW)], qy_v)
        pltpu.sync_copy(qz_h.at[pl.ds(qbase, _QW)], qz_v)

        for qg in range(_QW // 16):
            qxg = qx_v[pl.ds(qg * 16, 16)]
            qyg = qy_v[pl.ds(qg * 16, 16)]
            qzg = qz_v[pl.ds(qg * 16, 16)]

            def qbody(q, res, qxg=qxg, qyg=qyg, qzg=qzg):
                qxs = _splat(qxg, q)
                qys = _splat(qyg, q)
                qzs = _splat(qzg, q)

                def cbody(c, m):
                    for u in range(_UNROLL):
                        off = (c * _UNROLL + u) * 16
                        dx = tx_v[pl.ds(off, 16)] - qxs
                        dy = ty_v[pl.ds(off, 16)] - qys
                        dz = tz_v[pl.ds(off, 16)] - qzs
                        m = jnp.minimum(m, dx * dx + dy * dy + dz * dz)
                    return m

                m = lax.fori_loop(0, nchunk, cbody, inf16, unroll=1)
                for L in (1, 2, 4, 8):
                    perm = jnp.bitwise_xor(iota, L)
                    m = jnp.minimum(
                        m, m.at[perm].get(mode="promise_in_bounds")
                    )
                return jnp.where(iota == q, m, res)

            res = lax.fori_loop(0, 16, qbody, inf16)
            res_v[pl.ds(qg * 16, 16)] = res

        pltpu.sync_copy(res_v, out_h.at[pl.ds(qbase, _QW)])


def _sc_nn(qx, qy, qz, tx, ty, tz, B, NQ, NT):
    mesh = plsc.VectorSubcoreMesh(core_axis_name="c", subcore_axis_name="s")
    body = functools.partial(_nn_body, B, NT)
    f = pl.kernel(
        body,
        out_type=jax.ShapeDtypeStruct((B * NQ,), jnp.float32),
        mesh=mesh,
        scratch_types=[
            pltpu.VMEM((_QW,), jnp.float32),
            pltpu.VMEM((_QW,), jnp.float32),
            pltpu.VMEM((_QW,), jnp.float32),
            pltpu.VMEM((NT,), jnp.float32),
            pltpu.VMEM((NT,), jnp.float32),
            pltpu.VMEM((NT,), jnp.float32),
            pltpu.VMEM((_QW,), jnp.float32),
        ],
    )
    return f(qx, qy, qz, tx, ty, tz)


@jax.jit
def kernel(xyz1, xyz2):
    B, N, _ = xyz1.shape
    _, M, _ = xyz2.shape
    x1 = [xyz1[:, :, k].reshape(-1) for k in range(3)]
    x2 = [xyz2[:, :, k].reshape(-1) for k in range(3)]
    d1 = _sc_nn(*x1, *x2, B, N, M)
    d2 = _sc_nn(*x2, *x1, B, M, N)
    return (d1.reshape(B, N), d2.reshape(B, M))


# speedup vs baseline: 2.9746x; 2.9746x over previous
import functools

import jax
import jax.numpy as jnp
from jax import lax
from jax.experimental import pallas as pl
from jax.experimental.pallas import tpu as pltpu
from jax.experimental.pallas import tpu_sc as plsc

TN = 512
TM = 512
JH = 4
MS = 3584

_NC = 2
_NS = 16
_NW = _NC * _NS
_UNROLL = 4


def _chamfer_body(x1_ref, x2t_ref, d1_ref, d2_ref, x1b_ref, d1s_ref, d2s_ref):
    n = pl.program_id(1)
    m = pl.program_id(2)
    num_n = pl.num_programs(1)
    num_m = pl.num_programs(2)
    J = TM // 128
    R = TN // 8

    @pl.when(m == 0)
    def _():
        for k in range(3):
            x1b_ref[k] = jnp.broadcast_to(x1_ref[0, :, k : k + 1], (TN, 128))
        d1s_ref[...] = jnp.full((TN, 128), jnp.inf, jnp.float32)

    @pl.when((n == 0) & (m == 0))
    def _():
        d2s_ref[...] = jnp.full(d2s_ref.shape, jnp.inf, jnp.float32)

    for j0 in range(0, J, JH):
        t = [
            [
                jnp.broadcast_to(
                    x2t_ref[0, k : k + 1, pl.ds((j0 + j) * 128, 128)], (8, 128)
                )
                for k in range(3)
            ]
            for j in range(JH)
        ]
        colacc = [None] * JH
        for r in range(R):
            rs = pl.ds(r * 8, 8)
            a = [x1b_ref[k, rs, :] for k in range(3)]
            rowmin = None
            for j in range(JH):
                d0 = a[0] - t[j][0]
                d1 = a[1] - t[j][1]
                d2 = a[2] - t[j][2]
                acc = d0 * d0 + d1 * d1 + d2 * d2
                rowmin = acc if rowmin is None else jnp.minimum(rowmin, acc)
                colacc[j] = (
                    acc if colacc[j] is None else jnp.minimum(colacc[j], acc)
                )
            d1s_ref[rs, :] = jnp.minimum(d1s_ref[rs, :], rowmin)

        base = pl.multiple_of(m * TM, TM)
        for j in range(JH):
            sl = pl.ds(base + (j0 + j) * 128, 128)
            d2s_ref[:, sl] = jnp.minimum(d2s_ref[:, sl], colacc[j])

    @pl.when(m == num_m - 1)
    def _():
        d1_ref[0, 0, :] = jnp.min(d1s_ref[...], axis=1)

    @pl.when(n == num_n - 1)
    def _():
        d2_ref[0, 0, :] = jnp.min(
            d2s_ref[:, pl.ds(pl.multiple_of(m * TM, TM), TM)], axis=0
        )


def _tc_chamfer(xyz1, x2t):
    B, N, _ = xyz1.shape
    _, _, M = x2t.shape
    grid = (B, N // TN, M // TM)
    dist1, dist2 = pl.pallas_call(
        _chamfer_body,
        grid=grid,
        in_specs=[
            pl.BlockSpec((1, TN, 3), lambda b, n, m: (b, n, 0)),
            pl.BlockSpec((1, 3, TM), lambda b, n, m: (b, 0, m)),
        ],
        out_specs=[
            pl.BlockSpec((1, 1, TN), lambda b, n, m: (b, 0, n)),
            pl.BlockSpec((1, 1, TM), lambda b, n, m: (b, 0, m)),
        ],
        out_shape=[
            jax.ShapeDtypeStruct((B, 1, N), jnp.float32),
            jax.ShapeDtypeStruct((B, 1, M), jnp.float32),
        ],
        scratch_shapes=[
            pltpu.VMEM((3, TN, 128), jnp.float32),
            pltpu.VMEM((TN, 128), jnp.float32),
            pltpu.VMEM((8, M), jnp.float32),
        ],
        compiler_params=pltpu.CompilerParams(
            dimension_semantics=("arbitrary", "arbitrary", "arbitrary"),
        ),
    )(xyz1, x2t)
    return dist1[:, 0, :], dist2[:, 0, :]


def _splat(v, q):
    idx = jnp.full((16,), q, jnp.int32)
    return v.at[idx].get(mode="promise_in_bounds")


def _nn_body(B, QW, NT, qx_h, qy_h, qz_h, tx_h, ty_h, tz_h, out_h,
             qx_v, qy_v, qz_v, tx_v, ty_v, tz_v, res_v):
    wid = lax.axis_index("s") * _NC + lax.axis_index("c")
    iota = lax.iota(jnp.int32, 16)
    inf16 = jnp.full((16,), jnp.inf, jnp.float32)
    nchunk = NT // (16 * _UNROLL)

    for b in range(B):
        tbase = b * NT
        pltpu.sync_copy(tx_h.at[pl.ds(tbase, NT)], tx_v)
        pltpu.sync_copy(ty_h.at[pl.ds(tbase, NT)], ty_v)
        pltpu.sync_copy(tz_h.at[pl.ds(tbase, NT)], tz_v)
        qbase = b * (QW * _NW) + wid * QW
        pltpu.sync_copy(qx_h.at[pl.ds(qbase, QW)], qx_v)
        pltpu.sync_copy(qy_h.at[pl.ds(qbase, QW)], qy_v)
        pltpu.sync_copy(qz_h.at[pl.ds(qbase, QW)], qz_v)

        for qg in range(QW // 16):
            qxg = qx_v[pl.ds(qg * 16, 16)]
            qyg = qy_v[pl.ds(qg * 16, 16)]
            qzg = qz_v[pl.ds(qg * 16, 16)]

            def qbody(q, res, qxg=qxg, qyg=qyg, qzg=qzg):
                qxs = _splat(qxg, q)
                qys = _splat(qyg, q)
                qzs = _splat(qzg, q)

                def cbody(c, m):
                    for u in range(_UNROLL):
                        off = (c * _UNROLL + u) * 16
                        dx = tx_v[pl.ds(off, 16)] - qxs
                        dy = ty_v[pl.ds(off, 16)] - qys
                        dz = tz_v[pl.ds(off, 16)] - qzs
                        m = jnp.minimum(m, dx * dx + dy * dy + dz * dz)
                    return m

                m = lax.fori_loop(0, nchunk, cbody, inf16, unroll=1)
                for L in (1, 2, 4, 8):
                    perm = jnp.bitwise_xor(iota, L)
                    m = jnp.minimum(
                        m, m.at[perm].get(mode="promise_in_bounds")
                    )
                return jnp.where(iota == q, m, res)

            res = lax.fori_loop(0, 16, qbody, inf16)
            res_v[pl.ds(qg * 16, 16)] = res

        pltpu.sync_copy(res_v, out_h.at[pl.ds(qbase, QW)])


def _sc_nn(qx, qy, qz, tx, ty, tz, B, NQ, NT):
    qw = NQ // _NW
    mesh = plsc.VectorSubcoreMesh(core_axis_name="c", subcore_axis_name="s")
    body = functools.partial(_nn_body, B, qw, NT)
    f = pl.kernel(
        body,
        out_type=jax.ShapeDtypeStruct((B * NQ,), jnp.float32),
        mesh=mesh,
        scratch_types=[
            pltpu.VMEM((qw,), jnp.float32),
            pltpu.VMEM((qw,), jnp.float32),
            pltpu.VMEM((qw,), jnp.float32),
            pltpu.VMEM((NT,), jnp.float32),
            pltpu.VMEM((NT,), jnp.float32),
            pltpu.VMEM((NT,), jnp.float32),
            pltpu.VMEM((qw,), jnp.float32),
        ],
    )
    return f(qx, qy, qz, tx, ty, tz)


@jax.jit
def kernel(xyz1, xyz2):
    B, N, _ = xyz1.shape
    _, M, _ = xyz2.shape

    x2t_tc = jnp.transpose(xyz2[:, :MS, :], (0, 2, 1))
    d1_tc, d2_tc = _tc_chamfer(xyz1, x2t_tc)

    x1f = [xyz1[:, :, k].reshape(-1) for k in range(3)]
    x2s = [xyz2[:, MS:, k].reshape(-1) for k in range(3)]
    d1_sc = _sc_nn(*x1f, *x2s, B, N, M - MS)
    d2_sc = _sc_nn(*x2s, *x1f, B, M - MS, N)

    dist1 = jnp.minimum(d1_tc, d1_sc.reshape(B, N))
    dist2 = jnp.concatenate([d2_tc, d2_sc.reshape(B, M - MS)], axis=1)
    return (dist1, dist2)


# speedup vs baseline: 3.0067x; 1.0108x over previous
import functools

import jax
import jax.numpy as jnp
from jax import lax
from jax.experimental import pallas as pl
from jax.experimental.pallas import tpu as pltpu
from jax.experimental.pallas import tpu_sc as plsc

TN = 512
TM = 512
JH = 4
MS = 3584

_NC = 2
_NS = 16
_NW = _NC * _NS
_UNROLL = 4


def _chamfer_body(x1_ref, x2t_ref, d1_ref, d2_ref, x1b_ref, d1s_ref, d2s_ref):
    n = pl.program_id(1)
    m = pl.program_id(2)
    num_n = pl.num_programs(1)
    num_m = pl.num_programs(2)
    J = TM // 128
    R = TN // 8

    @pl.when(m == 0)
    def _():
        for k in range(3):
            x1b_ref[k] = jnp.broadcast_to(x1_ref[0, :, k : k + 1], (TN, 128))
        d1s_ref[...] = jnp.full((TN, 128), jnp.inf, jnp.float32)

    @pl.when((n == 0) & (m == 0))
    def _():
        d2s_ref[...] = jnp.full(d2s_ref.shape, jnp.inf, jnp.float32)

    for j0 in range(0, J, JH):
        t = [
            [
                jnp.broadcast_to(
                    x2t_ref[0, k : k + 1, pl.ds((j0 + j) * 128, 128)], (8, 128)
                )
                for k in range(3)
            ]
            for j in range(JH)
        ]
        colacc = [None] * JH
        for r in range(R):
            rs = pl.ds(r * 8, 8)
            a = [x1b_ref[k, rs, :] for k in range(3)]
            rowmin = None
            for j in range(JH):
                d0 = a[0] - t[j][0]
                d1 = a[1] - t[j][1]
                d2 = a[2] - t[j][2]
                acc = d0 * d0 + d1 * d1 + d2 * d2
                rowmin = acc if rowmin is None else jnp.minimum(rowmin, acc)
                colacc[j] = (
                    acc if colacc[j] is None else jnp.minimum(colacc[j], acc)
                )
            d1s_ref[rs, :] = jnp.minimum(d1s_ref[rs, :], rowmin)

        base = pl.multiple_of(m * TM, TM)
        for j in range(JH):
            sl = pl.ds(base + (j0 + j) * 128, 128)
            d2s_ref[:, sl] = jnp.minimum(d2s_ref[:, sl], colacc[j])

    @pl.when(m == num_m - 1)
    def _():
        d1_ref[0, 0, :] = jnp.min(d1s_ref[...], axis=1)

    @pl.when(n == num_n - 1)
    def _():
        d2_ref[0, 0, :] = jnp.min(
            d2s_ref[:, pl.ds(pl.multiple_of(m * TM, TM), TM)], axis=0
        )


def _tc_chamfer(xyz1, x2t):
    B, N, _ = xyz1.shape
    _, _, M = x2t.shape
    grid = (B, N // TN, M // TM)
    dist1, dist2 = pl.pallas_call(
        _chamfer_body,
        grid=grid,
        in_specs=[
            pl.BlockSpec((1, TN, 3), lambda b, n, m: (b, n, 0)),
            pl.BlockSpec((1, 3, TM), lambda b, n, m: (b, 0, m)),
        ],
        out_specs=[
            pl.BlockSpec((1, 1, TN), lambda b, n, m: (b, 0, n)),
            pl.BlockSpec((1, 1, TM), lambda b, n, m: (b, 0, m)),
        ],
        out_shape=[
            jax.ShapeDtypeStruct((B, 1, N), jnp.float32),
            jax.ShapeDtypeStruct((B, 1, M), jnp.float32),
        ],
        scratch_shapes=[
            pltpu.VMEM((3, TN, 128), jnp.float32),
            pltpu.VMEM((TN, 128), jnp.float32),
            pltpu.VMEM((8, M), jnp.float32),
        ],
        compiler_params=pltpu.CompilerParams(
            dimension_semantics=("arbitrary", "arbitrary", "arbitrary"),
        ),
    )(xyz1, x2t)
    return dist1[:, 0, :], dist2[:, 0, :]


def _splat(v, q):
    idx = jnp.full((16,), q, jnp.int32)
    return v.at[idx].get(mode="promise_in_bounds")


def _nn_section(B, QW, NT, wid, qx_h, qy_h, qz_h, tx_h, ty_h, tz_h, out_h,
                qx_v, qy_v, qz_v, tx_v, ty_v, tz_v, res_v):
    iota = lax.iota(jnp.int32, 16)
    inf16 = jnp.full((16,), jnp.inf, jnp.float32)
    nchunk = NT // (16 * _UNROLL)

    for b in range(B):
        tbase = b * NT
        pltpu.sync_copy(tx_h.at[pl.ds(tbase, NT)], tx_v)
        pltpu.sync_copy(ty_h.at[pl.ds(tbase, NT)], ty_v)
        pltpu.sync_copy(tz_h.at[pl.ds(tbase, NT)], tz_v)
        qbase = b * (QW * _NW) + wid * QW
        pltpu.sync_copy(qx_h.at[pl.ds(qbase, QW)], qx_v)
        pltpu.sync_copy(qy_h.at[pl.ds(qbase, QW)], qy_v)
        pltpu.sync_copy(qz_h.at[pl.ds(qbase, QW)], qz_v)

        for qg in range(QW // 16):
            qxg = qx_v[pl.ds(qg * 16, 16)]
            qyg = qy_v[pl.ds(qg * 16, 16)]
            qzg = qz_v[pl.ds(qg * 16, 16)]

            def qbody(q, res, qxg=qxg, qyg=qyg, qzg=qzg):
                qxs = _splat(qxg, q)
                qys = _splat(qyg, q)
                qzs = _splat(qzg, q)

                def cbody(c, m):
                    for u in range(_UNROLL):
                        off = (c * _UNROLL + u) * 16
                        dx = tx_v[pl.ds(off, 16)] - qxs
                        dy = ty_v[pl.ds(off, 16)] - qys
                        dz = tz_v[pl.ds(off, 16)] - qzs
                        m = jnp.minimum(m, dx * dx + dy * dy + dz * dz)
                    return m

                m = lax.fori_loop(0, nchunk, cbody, inf16, unroll=1)
                for L in (1, 2, 4, 8):
                    perm = jnp.bitwise_xor(iota, L)
                    m = jnp.minimum(
                        m, m.at[perm].get(mode="promise_in_bounds")
                    )
                return jnp.where(iota == q, m, res)

            res = lax.fori_loop(0, 16, qbody, inf16)
            res_v[pl.ds(qg * 16, 16)] = res

        pltpu.sync_copy(res_v, out_h.at[pl.ds(qbase, QW)])


def _sc_both_body(B, QW1, NT1, QW2, NT2, args1_h, args2_h, out1_h, out2_h,
                  *scratch):
    wid = lax.axis_index("s") * _NC + lax.axis_index("c")
    s1, s2 = scratch[:7], scratch[7:]
    _nn_section(B, QW1, NT1, wid, *args1_h, out1_h, *s1)
    _nn_section(B, QW2, NT2, wid, *args2_h, out2_h, *s2)


def _sc_both(x1f, x2s, B, N, MR):
    qw1 = N // _NW
    qw2 = MR // _NW
    mesh = plsc.VectorSubcoreMesh(core_axis_name="c", subcore_axis_name="s")

    def body(q1x, q1y, q1z, t1x, t1y, t1z, out1, out2, *scratch):
        _sc_both_body(
            B, qw1, MR, qw2, N,
            (q1x, q1y, q1z, t1x, t1y, t1z),
            (t1x, t1y, t1z, q1x, q1y, q1z),
            out1, out2, *scratch,
        )

    f = pl.kernel(
        body,
        out_type=[
            jax.ShapeDtypeStruct((B * N,), jnp.float32),
            jax.ShapeDtypeStruct((B * MR,), jnp.float32),
        ],
        mesh=mesh,
        scratch_types=[
            pltpu.VMEM((qw1,), jnp.float32),
            pltpu.VMEM((qw1,), jnp.float32),
            pltpu.VMEM((qw1,), jnp.float32),
            pltpu.VMEM((MR,), jnp.float32),
            pltpu.VMEM((MR,), jnp.float32),
            pltpu.VMEM((MR,), jnp.float32),
            pltpu.VMEM((qw1,), jnp.float32),
            pltpu.VMEM((qw2,), jnp.float32),
            pltpu.VMEM((qw2,), jnp.float32),
            pltpu.VMEM((qw2,), jnp.float32),
            pltpu.VMEM((N,), jnp.float32),
            pltpu.VMEM((N,), jnp.float32),
            pltpu.VMEM((N,), jnp.float32),
            pltpu.VMEM((qw2,), jnp.float32),
        ],
    )
    return f(*x1f, *x2s)


@jax.jit
def kernel(xyz1, xyz2):
    B, N, _ = xyz1.shape
    _, M, _ = xyz2.shape

    x2t_tc = jnp.transpose(xyz2[:, :MS, :], (0, 2, 1))
    d1_tc, d2_tc = _tc_chamfer(xyz1, x2t_tc)

    x1f = [xyz1[:, :, k].reshape(-1) for k in range(3)]
    x2s = [xyz2[:, MS:, k].reshape(-1) for k in range(3)]
    d1_sc, d2_sc = _sc_both(x1f, x2s, B, N, M - MS)

    dist1 = jnp.minimum(d1_tc, d1_sc.reshape(B, N))
    dist2 = jnp.concatenate([d2_tc, d2_sc.reshape(B, M - MS)], axis=1)
    return (dist1, dist2)


# speedup vs baseline: 3.0591x; 1.0174x over previous
import functools

import jax
import jax.numpy as jnp
from jax import lax
from jax.experimental import pallas as pl
from jax.experimental.pallas import tpu as pltpu
from jax.experimental.pallas import tpu_sc as plsc

TN = 512
TM = 512
JH = 4
MS = 4096

_NC = 2
_NS = 16
_NW = _NC * _NS
_UNROLL = 4


def _chamfer_body(x1_ref, x2t_ref, d1_ref, d2_ref, x1b_ref, d1s_ref, d2s_ref):
    n = pl.program_id(1)
    m = pl.program_id(2)
    num_n = pl.num_programs(1)
    num_m = pl.num_programs(2)
    J = TM // 128
    R = TN // 8

    @pl.when(m == 0)
    def _():
        for k in range(3):
            x1b_ref[k] = jnp.broadcast_to(x1_ref[0, :, k : k + 1], (TN, 128))
        d1s_ref[...] = jnp.full((TN, 128), jnp.inf, jnp.float32)

    @pl.when((n == 0) & (m == 0))
    def _():
        d2s_ref[...] = jnp.full(d2s_ref.shape, jnp.inf, jnp.float32)

    for j0 in range(0, J, JH):
        t = [
            [
                jnp.broadcast_to(
                    x2t_ref[0, k : k + 1, pl.ds((j0 + j) * 128, 128)], (8, 128)
                )
                for k in range(3)
            ]
            for j in range(JH)
        ]
        colacc = [None] * JH
        for r in range(R):
            rs = pl.ds(r * 8, 8)
            a = [x1b_ref[k, rs, :] for k in range(3)]
            rowmin = None
            for j in range(JH):
                d0 = a[0] - t[j][0]
                d1 = a[1] - t[j][1]
                d2 = a[2] - t[j][2]
                acc = d0 * d0 + d1 * d1 + d2 * d2
                rowmin = acc if rowmin is None else jnp.minimum(rowmin, acc)
                colacc[j] = (
                    acc if colacc[j] is None else jnp.minimum(colacc[j], acc)
                )
            d1s_ref[rs, :] = jnp.minimum(d1s_ref[rs, :], rowmin)

        base = pl.multiple_of(m * TM, TM)
        for j in range(JH):
            sl = pl.ds(base + (j0 + j) * 128, 128)
            d2s_ref[:, sl] = jnp.minimum(d2s_ref[:, sl], colacc[j])

    @pl.when(m == num_m - 1)
    def _():
        d1_ref[0, 0, :] = jnp.min(d1s_ref[...], axis=1)

    @pl.when(n == num_n - 1)
    def _():
        d2_ref[0, 0, :] = jnp.min(
            d2s_ref[:, pl.ds(pl.multiple_of(m * TM, TM), TM)], axis=0
        )


def _tc_chamfer(xyz1, x2t):
    B, N, _ = xyz1.shape
    _, _, M = x2t.shape
    grid = (B, N // TN, M // TM)
    dist1, dist2 = pl.pallas_call(
        _chamfer_body,
        grid=grid,
        in_specs=[
            pl.BlockSpec((1, TN, 3), lambda b, n, m: (b, n, 0)),
            pl.BlockSpec((1, 3, TM), lambda b, n, m: (b, 0, m)),
        ],
        out_specs=[
            pl.BlockSpec((1, 1, TN), lambda b, n, m: (b, 0, n)),
            pl.BlockSpec((1, 1, TM), lambda b, n, m: (b, 0, m)),
        ],
        out_shape=[
            jax.ShapeDtypeStruct((B, 1, N), jnp.float32),
            jax.ShapeDtypeStruct((B, 1, M), jnp.float32),
        ],
        scratch_shapes=[
            pltpu.VMEM((3, TN, 128), jnp.float32),
            pltpu.VMEM((TN, 128), jnp.float32),
            pltpu.VMEM((8, M), jnp.float32),
        ],
        compiler_params=pltpu.CompilerParams(
            dimension_semantics=("arbitrary", "arbitrary", "arbitrary"),
        ),
    )(xyz1, x2t)
    return dist1[:, 0, :], dist2[:, 0, :]


def _splat(v, q):
    idx = jnp.full((16,), q, jnp.int32)
    return v.at[idx].get(mode="promise_in_bounds")


def _nn_section(B, QW, NT, wid, qx_h, qy_h, qz_h, tx_h, ty_h, tz_h, out_h,
                qx_v, qy_v, qz_v, tx_v, ty_v, tz_v, res_v):
    iota = lax.iota(jnp.int32, 16)
    inf16 = jnp.full((16,), jnp.inf, jnp.float32)
    nchunk = NT // (16 * _UNROLL)

    for b in range(B):
        tbase = b * NT
        pltpu.sync_copy(tx_h.at[pl.ds(tbase, NT)], tx_v)
        pltpu.sync_copy(ty_h.at[pl.ds(tbase, NT)], ty_v)
        pltpu.sync_copy(tz_h.at[pl.ds(tbase, NT)], tz_v)
        qbase = b * (QW * _NW) + wid * QW
        pltpu.sync_copy(qx_h.at[pl.ds(qbase, QW)], qx_v)
        pltpu.sync_copy(qy_h.at[pl.ds(qbase, QW)], qy_v)
        pltpu.sync_copy(qz_h.at[pl.ds(qbase, QW)], qz_v)

        for qg in range(QW // 16):
            qxg = qx_v[pl.ds(qg * 16, 16)]
            qyg = qy_v[pl.ds(qg * 16, 16)]
            qzg = qz_v[pl.ds(qg * 16, 16)]

            def qbody(q, res, qxg=qxg, qyg=qyg, qzg=qzg):
                qxs = _splat(qxg, q)
                qys = _splat(qyg, q)
                qzs = _splat(qzg, q)

                def cbody(c, m):
                    for u in range(_UNROLL):
                        off = (c * _UNROLL + u) * 16
                        dx = tx_v[pl.ds(off, 16)] - qxs
                        dy = ty_v[pl.ds(off, 16)] - qys
                        dz = tz_v[pl.ds(off, 16)] - qzs
                        m = jnp.minimum(m, dx * dx + dy * dy + dz * dz)
                    return m

                m = lax.fori_loop(0, nchunk, cbody, inf16, unroll=1)
                for L in (1, 2, 4, 8):
                    perm = jnp.bitwise_xor(iota, L)
                    m = jnp.minimum(
                        m, m.at[perm].get(mode="promise_in_bounds")
                    )
                return jnp.where(iota == q, m, res)

            res = lax.fori_loop(0, 16, qbody, inf16)
            res_v[pl.ds(qg * 16, 16)] = res

        pltpu.sync_copy(res_v, out_h.at[pl.ds(qbase, QW)])


def _sc_both_body(B, QW1, NT1, QW2, NT2, args1_h, args2_h, out1_h, out2_h,
                  *scratch):
    wid = lax.axis_index("s") * _NC + lax.axis_index("c")
    s1, s2 = scratch[:7], scratch[7:]
    _nn_section(B, QW1, NT1, wid, *args1_h, out1_h, *s1)
    _nn_section(B, QW2, NT2, wid, *args2_h, out2_h, *s2)


def _sc_both(x1f, x2s, B, N, MR):
    qw1 = N // _NW
    qw2 = MR // _NW
    mesh = plsc.VectorSubcoreMesh(core_axis_name="c", subcore_axis_name="s")

    def body(q1x, q1y, q1z, t1x, t1y, t1z, out1, out2, *scratch):
        _sc_both_body(
            B, qw1, MR, qw2, N,
            (q1x, q1y, q1z, t1x, t1y, t1z),
            (t1x, t1y, t1z, q1x, q1y, q1z),
            out1, out2, *scratch,
        )

    f = pl.kernel(
        body,
        out_type=[
            jax.ShapeDtypeStruct((B * N,), jnp.float32),
            jax.ShapeDtypeStruct((B * MR,), jnp.float32),
        ],
        mesh=mesh,
        scratch_types=[
            pltpu.VMEM((qw1,), jnp.float32),
            pltpu.VMEM((qw1,), jnp.float32),
            pltpu.VMEM((qw1,), jnp.float32),
            pltpu.VMEM((MR,), jnp.float32),
            pltpu.VMEM((MR,), jnp.float32),
            pltpu.VMEM((MR,), jnp.float32),
            pltpu.VMEM((qw1,), jnp.float32),
            pltpu.VMEM((qw2,), jnp.float32),
            pltpu.VMEM((qw2,), jnp.float32),
            pltpu.VMEM((qw2,), jnp.float32),
            pltpu.VMEM((N,), jnp.float32),
            pltpu.VMEM((N,), jnp.float32),
            pltpu.VMEM((N,), jnp.float32),
            pltpu.VMEM((qw2,), jnp.float32),
        ],
    )
    return f(*x1f, *x2s)


@jax.jit
def kernel(xyz1, xyz2):
    B, N, _ = xyz1.shape
    _, M, _ = xyz2.shape

    x2t_tc = jnp.transpose(xyz2[:, :MS, :], (0, 2, 1))
    d1_tc, d2_tc = _tc_chamfer(xyz1, x2t_tc)

    return (d1_tc, d2_tc)


# speedup vs baseline: 3.7038x; 1.2107x over previous
import functools

import jax
import jax.numpy as jnp
from jax import lax
from jax.experimental import pallas as pl
from jax.experimental.pallas import tpu as pltpu
from jax.experimental.pallas import tpu_sc as plsc

TN = 512
TM = 896
JH = 4
MS = 3584

_NC = 2
_NS = 16
_NW = _NC * _NS
_UNROLL = 4


def _chamfer_body(x1_ref, x2t_ref, d1_ref, d2_ref, x1b_ref, d1s_ref, d2s_ref):
    n = pl.program_id(1)
    m = pl.program_id(2)
    num_n = pl.num_programs(1)
    num_m = pl.num_programs(2)
    J = TM // 128
    R = TN // 8

    @pl.when(m == 0)
    def _():
        for k in range(3):
            x1b_ref[k] = jnp.broadcast_to(x1_ref[0, :, k : k + 1], (TN, 128))
        d1s_ref[...] = jnp.full((TN, 128), jnp.inf, jnp.float32)

    @pl.when((n == 0) & (m == 0))
    def _():
        d2s_ref[...] = jnp.full(d2s_ref.shape, jnp.inf, jnp.float32)

    for j0 in range(0, J, JH):
        jh = min(JH, J - j0)
        t = [
            [
                jnp.broadcast_to(
                    x2t_ref[0, k : k + 1, pl.ds((j0 + j) * 128, 128)], (8, 128)
                )
                for k in range(3)
            ]
            for j in range(jh)
        ]
        colacc = [None] * jh
        for r in range(R):
            rs = pl.ds(r * 8, 8)
            a = [x1b_ref[k, rs, :] for k in range(3)]
            rowmin = None
            for j in range(jh):
                d0 = a[0] - t[j][0]
                d1 = a[1] - t[j][1]
                d2 = a[2] - t[j][2]
                acc = d0 * d0 + d1 * d1 + d2 * d2
                rowmin = acc if rowmin is None else jnp.minimum(rowmin, acc)
                colacc[j] = (
                    acc if colacc[j] is None else jnp.minimum(colacc[j], acc)
                )
            d1s_ref[rs, :] = jnp.minimum(d1s_ref[rs, :], rowmin)

        base = pl.multiple_of(m * TM, 128)
        for j in range(jh):
            sl = pl.ds(base + (j0 + j) * 128, 128)
            d2s_ref[:, sl] = jnp.minimum(d2s_ref[:, sl], colacc[j])

    @pl.when(m == num_m - 1)
    def _():
        d1_ref[0, 0, :] = jnp.min(d1s_ref[...], axis=1)

    @pl.when(n == num_n - 1)
    def _():
        d2_ref[0, 0, :] = jnp.min(
            d2s_ref[:, pl.ds(pl.multiple_of(m * TM, 128), TM)], axis=0
        )


def _tc_chamfer(xyz1, x2t):
    B, N, _ = xyz1.shape
    _, _, M = x2t.shape
    grid = (B, N // TN, M // TM)
    dist1, dist2 = pl.pallas_call(
        _chamfer_body,
        grid=grid,
        in_specs=[
            pl.BlockSpec((1, TN, 3), lambda b, n, m: (b, n, 0)),
            pl.BlockSpec((1, 3, TM), lambda b, n, m: (b, 0, m)),
        ],
        out_specs=[
            pl.BlockSpec((1, 1, TN), lambda b, n, m: (b, 0, n)),
            pl.BlockSpec((1, 1, TM), lambda b, n, m: (b, 0, m)),
        ],
        out_shape=[
            jax.ShapeDtypeStruct((B, 1, N), jnp.float32),
            jax.ShapeDtypeStruct((B, 1, M), jnp.float32),
        ],
        scratch_shapes=[
            pltpu.VMEM((3, TN, 128), jnp.float32),
            pltpu.VMEM((TN, 128), jnp.float32),
            pltpu.VMEM((8, M), jnp.float32),
        ],
        compiler_params=pltpu.CompilerParams(
            dimension_semantics=("arbitrary", "arbitrary", "arbitrary"),
        ),
    )(xyz1, x2t)
    return dist1[:, 0, :], dist2[:, 0, :]


def _splat(v, q):
    idx = jnp.full((16,), q, jnp.int32)
    return v.at[idx].get(mode="promise_in_bounds")


def _nn_section(B, QW, NT, wid, qx_h, qy_h, qz_h, tx_h, ty_h, tz_h, out_h,
                qx_v, qy_v, qz_v, tx_v, ty_v, tz_v, res_v):
    iota = lax.iota(jnp.int32, 16)
    inf16 = jnp.full((16,), jnp.inf, jnp.float32)
    nchunk = NT // (16 * _UNROLL)

    for b in range(B):
        tbase = b * NT
        pltpu.sync_copy(tx_h.at[pl.ds(tbase, NT)], tx_v)
        pltpu.sync_copy(ty_h.at[pl.ds(tbase, NT)], ty_v)
        pltpu.sync_copy(tz_h.at[pl.ds(tbase, NT)], tz_v)
        qbase = b * (QW * _NW) + wid * QW
        pltpu.sync_copy(qx_h.at[pl.ds(qbase, QW)], qx_v)
        pltpu.sync_copy(qy_h.at[pl.ds(qbase, QW)], qy_v)
        pltpu.sync_copy(qz_h.at[pl.ds(qbase, QW)], qz_v)

        for qg in range(QW // 16):
            qxg = qx_v[pl.ds(qg * 16, 16)]
            qyg = qy_v[pl.ds(qg * 16, 16)]
            qzg = qz_v[pl.ds(qg * 16, 16)]

            def qbody(q, res, qxg=qxg, qyg=qyg, qzg=qzg):
                qxs = _splat(qxg, q)
                qys = _splat(qyg, q)
                qzs = _splat(qzg, q)

                def cbody(c, m):
                    for u in range(_UNROLL):
                        off = (c * _UNROLL + u) * 16
                        dx = tx_v[pl.ds(off, 16)] - qxs
                        dy = ty_v[pl.ds(off, 16)] - qys
                        dz = tz_v[pl.ds(off, 16)] - qzs
                        m = jnp.minimum(m, dx * dx + dy * dy + dz * dz)
                    return m

                m = lax.fori_loop(0, nchunk, cbody, inf16, unroll=1)
                for L in (1, 2, 4, 8):
                    perm = jnp.bitwise_xor(iota, L)
                    m = jnp.minimum(
                        m, m.at[perm].get(mode="promise_in_bounds")
                    )
                return jnp.where(iota == q, m, res)

            res = lax.fori_loop(0, 16, qbody, inf16)
            res_v[pl.ds(qg * 16, 16)] = res

        pltpu.sync_copy(res_v, out_h.at[pl.ds(qbase, QW)])


def _sc_both_body(B, QW1, NT1, QW2, NT2, args1_h, args2_h, out1_h, out2_h,
                  *scratch):
    wid = lax.axis_index("s") * _NC + lax.axis_index("c")
    s1, s2 = scratch[:7], scratch[7:]
    _nn_section(B, QW1, NT1, wid, *args1_h, out1_h, *s1)
    _nn_section(B, QW2, NT2, wid, *args2_h, out2_h, *s2)


def _sc_both(x1f, x2s, B, N, MR):
    qw1 = N // _NW
    qw2 = MR // _NW
    mesh = plsc.VectorSubcoreMesh(core_axis_name="c", subcore_axis_name="s")

    def body(q1x, q1y, q1z, t1x, t1y, t1z, out1, out2, *scratch):
        _sc_both_body(
            B, qw1, MR, qw2, N,
            (q1x, q1y, q1z, t1x, t1y, t1z),
            (t1x, t1y, t1z, q1x, q1y, q1z),
            out1, out2, *scratch,
        )

    f = pl.kernel(
        body,
        out_type=[
            jax.ShapeDtypeStruct((B * N,), jnp.float32),
            jax.ShapeDtypeStruct((B * MR,), jnp.float32),
        ],
        mesh=mesh,
        scratch_types=[
            pltpu.VMEM((qw1,), jnp.float32),
            pltpu.VMEM((qw1,), jnp.float32),
            pltpu.VMEM((qw1,), jnp.float32),
            pltpu.VMEM((MR,), jnp.float32),
            pltpu.VMEM((MR,), jnp.float32),
            pltpu.VMEM((MR,), jnp.float32),
            pltpu.VMEM((qw1,), jnp.float32),
            pltpu.VMEM((qw2,), jnp.float32),
            pltpu.VMEM((qw2,), jnp.float32),
            pltpu.VMEM((qw2,), jnp.float32),
            pltpu.VMEM((N,), jnp.float32),
            pltpu.VMEM((N,), jnp.float32),
            pltpu.VMEM((N,), jnp.float32),
            pltpu.VMEM((qw2,), jnp.float32),
        ],
    )
    return f(*x1f, *x2s)


@jax.jit
def kernel(xyz1, xyz2):
    B, N, _ = xyz1.shape
    _, M, _ = xyz2.shape

    x2t_tc = jnp.transpose(xyz2[:, :MS, :], (0, 2, 1))
    d1_tc, d2_tc = _tc_chamfer(xyz1, x2t_tc)

    x1f = [xyz1[:, :, k].reshape(-1) for k in range(3)]
    x2s = [xyz2[:, MS:, k].reshape(-1) for k in range(3)]
    d1_sc, d2_sc = _sc_both(x1f, x2s, B, N, M - MS)

    dist1 = jnp.minimum(d1_tc, d1_sc.reshape(B, N))
    dist2 = jnp.concatenate([d2_tc, d2_sc.reshape(B, M - MS)], axis=1)
    return (dist1, dist2)


# speedup vs baseline: 3.8693x; 1.0447x over previous
import functools

import jax
import jax.numpy as jnp
from jax import lax
from jax.experimental import pallas as pl
from jax.experimental.pallas import tpu as pltpu
from jax.experimental.pallas import tpu_sc as plsc

TN = 512
TM = 1792
JH = 4
MS = 3584

_NC = 2
_NS = 16
_NW = _NC * _NS
_UNROLL = 4


def _chamfer_body(x1_ref, x2t_ref, d1_ref, d2_ref, x1b_ref, d1s_ref, d2s_ref):
    n = pl.program_id(1)
    m = pl.program_id(2)
    num_n = pl.num_programs(1)
    num_m = pl.num_programs(2)
    J = TM // 128
    R = TN // 8

    @pl.when(m == 0)
    def _():
        for k in range(3):
            x1b_ref[k] = jnp.broadcast_to(x1_ref[0, :, k : k + 1], (TN, 128))
        d1s_ref[...] = jnp.full((TN, 128), jnp.inf, jnp.float32)

    @pl.when((n == 0) & (m == 0))
    def _():
        d2s_ref[...] = jnp.full(d2s_ref.shape, jnp.inf, jnp.float32)

    for j0 in range(0, J, JH):
        jh = min(JH, J - j0)
        t = [
            [
                jnp.broadcast_to(
                    x2t_ref[0, k : k + 1, pl.ds((j0 + j) * 128, 128)], (8, 128)
                )
                for k in range(3)
            ]
            for j in range(jh)
        ]
        colacc = [None] * jh
        for r in range(R):
            rs = pl.ds(r * 8, 8)
            a = [x1b_ref[k, rs, :] for k in range(3)]
            rowmin = None
            for j in range(jh):
                d0 = a[0] - t[j][0]
                d1 = a[1] - t[j][1]
                d2 = a[2] - t[j][2]
                acc = d0 * d0 + d1 * d1 + d2 * d2
                rowmin = acc if rowmin is None else jnp.minimum(rowmin, acc)
                colacc[j] = (
                    acc if colacc[j] is None else jnp.minimum(colacc[j], acc)
                )
            d1s_ref[rs, :] = jnp.minimum(d1s_ref[rs, :], rowmin)

        base = pl.multiple_of(m * TM, 128)
        for j in range(jh):
            sl = pl.ds(base + (j0 + j) * 128, 128)
            d2s_ref[:, sl] = jnp.minimum(d2s_ref[:, sl], colacc[j])

    @pl.when(m == num_m - 1)
    def _():
        d1_ref[0, 0, :] = jnp.min(d1s_ref[...], axis=1)

    @pl.when(n == num_n - 1)
    def _():
        d2_ref[0, 0, :] = jnp.min(
            d2s_ref[:, pl.ds(pl.multiple_of(m * TM, 128), TM)], axis=0
        )


def _tc_chamfer(xyz1, x2t):
    B, N, _ = xyz1.shape
    _, _, M = x2t.shape
    grid = (B, N // TN, M // TM)
    dist1, dist2 = pl.pallas_call(
        _chamfer_body,
        grid=grid,
        in_specs=[
            pl.BlockSpec((1, TN, 3), lambda b, n, m: (b, n, 0)),
            pl.BlockSpec((1, 3, TM), lambda b, n, m: (b, 0, m)),
        ],
        out_specs=[
            pl.BlockSpec((1, 1, TN), lambda b, n, m: (b, 0, n)),
            pl.BlockSpec((1, 1, TM), lambda b, n, m: (b, 0, m)),
        ],
        out_shape=[
            jax.ShapeDtypeStruct((B, 1, N), jnp.float32),
            jax.ShapeDtypeStruct((B, 1, M), jnp.float32),
        ],
        scratch_shapes=[
            pltpu.VMEM((3, TN, 128), jnp.float32),
            pltpu.VMEM((TN, 128), jnp.float32),
            pltpu.VMEM((8, M), jnp.float32),
        ],
        compiler_params=pltpu.CompilerParams(
            dimension_semantics=("arbitrary", "arbitrary", "arbitrary"),
        ),
    )(xyz1, x2t)
    return dist1[:, 0, :], dist2[:, 0, :]


def _splat(v, q):
    idx = jnp.full((16,), q, jnp.int32)
    return v.at[idx].get(mode="promise_in_bounds")


def _nn_section(B, QW, NT, wid, qx_h, qy_h, qz_h, tx_h, ty_h, tz_h, out_h,
                qx_v, qy_v, qz_v, tx_v, ty_v, tz_v, res_v):
    iota = lax.iota(jnp.int32, 16)
    inf16 = jnp.full((16,), jnp.inf, jnp.float32)
    nchunk = NT // (16 * _UNROLL)

    for b in range(B):
        tbase = b * NT
        pltpu.sync_copy(tx_h.at[pl.ds(tbase, NT)], tx_v)
        pltpu.sync_copy(ty_h.at[pl.ds(tbase, NT)], ty_v)
        pltpu.sync_copy(tz_h.at[pl.ds(tbase, NT)], tz_v)
        qbase = b * (QW * _NW) + wid * QW
        pltpu.sync_copy(qx_h.at[pl.ds(qbase, QW)], qx_v)
        pltpu.sync_copy(qy_h.at[pl.ds(qbase, QW)], qy_v)
        pltpu.sync_copy(qz_h.at[pl.ds(qbase, QW)], qz_v)

        for qg in range(QW // 16):
            qxg = qx_v[pl.ds(qg * 16, 16)]
            qyg = qy_v[pl.ds(qg * 16, 16)]
            qzg = qz_v[pl.ds(qg * 16, 16)]

            def qbody(q, res, qxg=qxg, qyg=qyg, qzg=qzg):
                qxs = _splat(qxg, q)
                qys = _splat(qyg, q)
                qzs = _splat(qzg, q)

                def cbody(c, m):
                    for u in range(_UNROLL):
                        off = (c * _UNROLL + u) * 16
                        dx = tx_v[pl.ds(off, 16)] - qxs
                        dy = ty_v[pl.ds(off, 16)] - qys
                        dz = tz_v[pl.ds(off, 16)] - qzs
                        m = jnp.minimum(m, dx * dx + dy * dy + dz * dz)
                    return m

                m = lax.fori_loop(0, nchunk, cbody, inf16, unroll=1)
                for L in (1, 2, 4, 8):
                    perm = jnp.bitwise_xor(iota, L)
                    m = jnp.minimum(
                        m, m.at[perm].get(mode="promise_in_bounds")
                    )
                return jnp.where(iota == q, m, res)

            res = lax.fori_loop(0, 16, qbody, inf16)
            res_v[pl.ds(qg * 16, 16)] = res

        pltpu.sync_copy(res_v, out_h.at[pl.ds(qbase, QW)])


def _sc_both_body(B, QW1, NT1, QW2, NT2, args1_h, args2_h, out1_h, out2_h,
                  *scratch):
    wid = lax.axis_index("s") * _NC + lax.axis_index("c")
    s1, s2 = scratch[:7], scratch[7:]
    _nn_section(B, QW1, NT1, wid, *args1_h, out1_h, *s1)
    _nn_section(B, QW2, NT2, wid, *args2_h, out2_h, *s2)


def _sc_both(x1f, x2s, B, N, MR):
    qw1 = N // _NW
    qw2 = MR // _NW
    mesh = plsc.VectorSubcoreMesh(core_axis_name="c", subcore_axis_name="s")

    def body(q1x, q1y, q1z, t1x, t1y, t1z, out1, out2, *scratch):
        _sc_both_body(
            B, qw1, MR, qw2, N,
            (q1x, q1y, q1z, t1x, t1y, t1z),
            (t1x, t1y, t1z, q1x, q1y, q1z),
            out1, out2, *scratch,
        )

    f = pl.kernel(
        body,
        out_type=[
            jax.ShapeDtypeStruct((B * N,), jnp.float32),
            jax.ShapeDtypeStruct((B * MR,), jnp.float32),
        ],
        mesh=mesh,
        scratch_types=[
            pltpu.VMEM((qw1,), jnp.float32),
            pltpu.VMEM((qw1,), jnp.float32),
            pltpu.VMEM((qw1,), jnp.float32),
            pltpu.VMEM((MR,), jnp.float32),
            pltpu.VMEM((MR,), jnp.float32),
            pltpu.VMEM((MR,), jnp.float32),
            pltpu.VMEM((qw1,), jnp.float32),
            pltpu.VMEM((qw2,), jnp.float32),
            pltpu.VMEM((qw2,), jnp.float32),
            pltpu.VMEM((qw2,), jnp.float32),
            pltpu.VMEM((N,), jnp.float32),
            pltpu.VMEM((N,), jnp.float32),
            pltpu.VMEM((N,), jnp.float32),
            pltpu.VMEM((qw2,), jnp.float32),
        ],
    )
    return f(*x1f, *x2s)


@jax.jit
def kernel(xyz1, xyz2):
    B, N, _ = xyz1.shape
    _, M, _ = xyz2.shape

    x2t_tc = jnp.transpose(xyz2[:, :MS, :], (0, 2, 1))
    d1_tc, d2_tc = _tc_chamfer(xyz1, x2t_tc)

    x1f = [xyz1[:, :, k].reshape(-1) for k in range(3)]
    x2s = [xyz2[:, MS:, k].reshape(-1) for k in range(3)]
    d1_sc, d2_sc = _sc_both(x1f, x2s, B, N, M - MS)

    dist1 = jnp.minimum(d1_tc, d1_sc.reshape(B, N))
    dist2 = jnp.concatenate([d2_tc, d2_sc.reshape(B, M - MS)], axis=1)
    return (dist1, dist2)


# speedup vs baseline: 4.2312x; 1.0935x over previous
import jax
import jax.numpy as jnp
from jax.experimental import pallas as pl
from jax.experimental.pallas import tpu as pltpu

TN = 512
TM = 2048
JH = 4


def _chamfer_body(x1_ref, x2t_ref, d1_ref, d2_ref, x1b_ref, d1s_ref, d2s_ref):
    n = pl.program_id(1)
    m = pl.program_id(2)
    num_n = pl.num_programs(1)
    num_m = pl.num_programs(2)
    J = TM // 128
    R = TN // 8

    @pl.when(m == 0)
    def _():
        for k in range(3):
            x1b_ref[k] = jnp.broadcast_to(x1_ref[0, :, k : k + 1], (TN, 128))
        d1s_ref[...] = jnp.full((TN, 128), jnp.inf, jnp.float32)

    @pl.when((n == 0) & (m == 0))
    def _():
        d2s_ref[...] = jnp.full(d2s_ref.shape, jnp.inf, jnp.float32)

    for j0 in range(0, J, JH):
        t = [
            [
                jnp.broadcast_to(
                    x2t_ref[0, k : k + 1, pl.ds((j0 + j) * 128, 128)], (8, 128)
                )
                for k in range(3)
            ]
            for j in range(JH)
        ]
        colacc = [None] * JH
        for r in range(R):
            rs = pl.ds(r * 8, 8)
            a = [x1b_ref[k, rs, :] for k in range(3)]
            rowmin = None
            for j in range(JH):
                d0 = a[0] - t[j][0]
                d1 = a[1] - t[j][1]
                d2 = a[2] - t[j][2]
                acc = d0 * d0 + d1 * d1 + d2 * d2
                rowmin = acc if rowmin is None else jnp.minimum(rowmin, acc)
                colacc[j] = (
                    acc if colacc[j] is None else jnp.minimum(colacc[j], acc)
                )
            d1s_ref[rs, :] = jnp.minimum(d1s_ref[rs, :], rowmin)

        base = pl.multiple_of(m * TM, TM)
        for j in range(JH):
            sl = pl.ds(base + (j0 + j) * 128, 128)
            d2s_ref[:, sl] = jnp.minimum(d2s_ref[:, sl], colacc[j])

    @pl.when(m == num_m - 1)
    def _():
        d1_ref[0, 0, :] = jnp.min(d1s_ref[...], axis=1)

    @pl.when(n == num_n - 1)
    def _():
        d2_ref[0, 0, :] = jnp.min(
            d2s_ref[:, pl.ds(pl.multiple_of(m * TM, TM), TM)], axis=0
        )


@jax.jit
def kernel(xyz1, xyz2):
    B, N, _ = xyz1.shape
    _, M, _ = xyz2.shape
    x2t = jnp.transpose(xyz2, (0, 2, 1))

    grid = (B, N // TN, M // TM)
    dist1, dist2 = pl.pallas_call(
        _chamfer_body,
        grid=grid,
        in_specs=[
            pl.BlockSpec((1, TN, 3), lambda b, n, m: (b, n, 0)),
            pl.BlockSpec((1, 3, TM), lambda b, n, m: (b, 0, m)),
        ],
        out_specs=[
            pl.BlockSpec((1, 1, TN), lambda b, n, m: (b, 0, n)),
            pl.BlockSpec((1, 1, TM), lambda b, n, m: (b, 0, m)),
        ],
        out_shape=[
            jax.ShapeDtypeStruct((B, 1, N), jnp.float32),
            jax.ShapeDtypeStruct((B, 1, M), jnp.float32),
        ],
        scratch_shapes=[
            pltpu.VMEM((3, TN, 128), jnp.float32),
            pltpu.VMEM((TN, 128), jnp.float32),
            pltpu.VMEM((8, M), jnp.float32),
        ],
        compiler_params=pltpu.CompilerParams(
            dimension_semantics=("arbitrary", "arbitrary", "arbitrary"),
        ),
    )(xyz1, x2t)
    return (dist1[:, 0, :], dist2[:, 0, :])


# speedup vs baseline: 4.5062x; 1.0650x over previous
import jax
import jax.numpy as jnp
from jax.experimental import pallas as pl
from jax.experimental.pallas import tpu as pltpu

TN = 512
TM = 4096
JH = 4


def _chamfer_body(x1_ref, x2t_ref, d1_ref, d2_ref, x1b_ref, d1s_ref, d2s_ref):
    n = pl.program_id(1)
    m = pl.program_id(2)
    num_n = pl.num_programs(1)
    num_m = pl.num_programs(2)
    J = TM // 128
    R = TN // 8

    @pl.when(m == 0)
    def _():
        for k in range(3):
            x1b_ref[k] = jnp.broadcast_to(x1_ref[0, :, k : k + 1], (TN, 128))
        d1s_ref[...] = jnp.full((TN, 128), jnp.inf, jnp.float32)

    @pl.when((n == 0) & (m == 0))
    def _():
        d2s_ref[...] = jnp.full(d2s_ref.shape, jnp.inf, jnp.float32)

    for j0 in range(0, J, JH):
        t = [
            [
                jnp.broadcast_to(
                    x2t_ref[0, k : k + 1, pl.ds((j0 + j) * 128, 128)], (8, 128)
                )
                for k in range(3)
            ]
            for j in range(JH)
        ]
        colacc = [None] * JH
        for r in range(R):
            rs = pl.ds(r * 8, 8)
            a = [x1b_ref[k, rs, :] for k in range(3)]
            rowmin = None
            for j in range(JH):
                d0 = a[0] - t[j][0]
                d1 = a[1] - t[j][1]
                d2 = a[2] - t[j][2]
                acc = d0 * d0 + d1 * d1 + d2 * d2
                rowmin = acc if rowmin is None else jnp.minimum(rowmin, acc)
                colacc[j] = (
                    acc if colacc[j] is None else jnp.minimum(colacc[j], acc)
                )
            d1s_ref[rs, :] = jnp.minimum(d1s_ref[rs, :], rowmin)

        base = pl.multiple_of(m * TM, TM)
        for j in range(JH):
            sl = pl.ds(base + (j0 + j) * 128, 128)
            d2s_ref[:, sl] = jnp.minimum(d2s_ref[:, sl], colacc[j])

    @pl.when(m == num_m - 1)
    def _():
        d1_ref[0, 0, :] = jnp.min(d1s_ref[...], axis=1)

    @pl.when(n == num_n - 1)
    def _():
        d2_ref[0, 0, :] = jnp.min(
            d2s_ref[:, pl.ds(pl.multiple_of(m * TM, TM), TM)], axis=0
        )


@jax.jit
def kernel(xyz1, xyz2):
    B, N, _ = xyz1.shape
    _, M, _ = xyz2.shape
    x2t = jnp.transpose(xyz2, (0, 2, 1))

    grid = (B, N // TN, M // TM)
    dist1, dist2 = pl.pallas_call(
        _chamfer_body,
        grid=grid,
        in_specs=[
            pl.BlockSpec((1, TN, 3), lambda b, n, m: (b, n, 0)),
            pl.BlockSpec((1, 3, TM), lambda b, n, m: (b, 0, m)),
        ],
        out_specs=[
            pl.BlockSpec((1, 1, TN), lambda b, n, m: (b, 0, n)),
            pl.BlockSpec((1, 1, TM), lambda b, n, m: (b, 0, m)),
        ],
        out_shape=[
            jax.ShapeDtypeStruct((B, 1, N), jnp.float32),
            jax.ShapeDtypeStruct((B, 1, M), jnp.float32),
        ],
        scratch_shapes=[
            pltpu.VMEM((3, TN, 128), jnp.float32),
            pltpu.VMEM((TN, 128), jnp.float32),
            pltpu.VMEM((8, M), jnp.float32),
        ],
        compiler_params=pltpu.CompilerParams(
            dimension_semantics=("arbitrary", "arbitrary", "arbitrary"),
        ),
    )(xyz1, x2t)
    return (dist1[:, 0, :], dist2[:, 0, :])
